# baseline (speedup 1.0000x reference)
import functools

import jax
import jax.numpy as jnp
from jax import lax
from jax.experimental import pallas as pl
from jax.experimental.pallas import tpu as pltpu
from jax.experimental.pallas import tpu_sc as plsc

_SIZE = 4096
_NC = 2
_NS = 16
_NW = _NC * _NS
_L = 16
_R = 8
_NBUF = 2


def _bspline_call(x, a_t, b_t):
    b_dim, one, h, w = x.shape
    rows = b_dim * h
    rows_w = rows // _NW
    nch = rows_w // _R
    mesh = plsc.VectorSubcoreMesh(core_axis_name="c", subcore_axis_name="s")

    @functools.partial(
        pl.kernel,
        out_type=jax.ShapeDtypeStruct(x.shape, jnp.float32),
        mesh=mesh,
        scratch_types=[
            pltpu.VMEM((_SIZE,), jnp.float32),
            pltpu.VMEM((_SIZE,), jnp.float32),
            pltpu.VMEM((_NBUF, _R, 2048), jnp.float32),
            pltpu.VMEM((_NBUF, _R, 2048), jnp.float32),
        ] + [pltpu.SemaphoreType.DMA] * (2 * _NBUF),
        compiler_params=pltpu.CompilerParams(
            needs_layout_passes=False, use_tc_tiling_on_sc=True),
    )
    def k(x_hbm, a_hbm, b_hbm, out_hbm, a_v, b_v, in_v, out_v, *sems):
        wid = lax.axis_index("s") * _NC + lax.axis_index("c")
        row0 = wid * rows_w
        batch = row0 // h
        brow0 = row0 % h
        sems_in = sems[:_NBUF]
        sems_out = sems[_NBUF:]

        pltpu.sync_copy(a_hbm, a_v)
        pltpu.sync_copy(b_hbm, b_v)

        def in_cp(g, b):
            return pltpu.make_async_copy(
                x_hbm.at[batch, 0, pl.ds(brow0 + g * _R, _R), :],
                in_v.at[b], sems_in[b])

        def out_cp(g, b):
            return pltpu.make_async_copy(
                out_v.at[b],
                out_hbm.at[batch, 0, pl.ds(brow0 + g * _R, _R), :],
                sems_out[b])

        def compute(b):
            @plsc.parallel_loop(0, w, step=_L, unroll=2)
            def _vec(j):
                for r in range(_R):
                    xv = in_v[b, r, pl.ds(j, _L)]
                    t = xv * 1000.0 + 2048.0
                    t = jnp.minimum(jnp.maximum(t, 0.0), 4095.0)
                    i = t.astype(jnp.int32)
                    a = plsc.load_gather(a_v, [i])
                    bb = plsc.load_gather(b_v, [i])
                    out_v[b, r, pl.ds(j, _L)] = a * t + bb

        for b in range(_NBUF):
            in_cp(b, b).start()

        def ring(p, carry):
            for b in range(_NBUF):
                g = p * _NBUF + b
                in_cp(g, b).wait()

                @pl.when(g >= _NBUF)
                def _wait_out():
                    out_cp(g - _NBUF, b).wait()

                compute(b)
                out_cp(g, b).start()

                @pl.when(g + _NBUF < nch)
                def _next_in():
                    in_cp(g + _NBUF, b).start()

            return carry

        lax.fori_loop(0, nch // _NBUF, ring, 0)
        for b in range(_NBUF):
            out_cp(nch - _NBUF + b, b).wait()

    return k(x, a_t, b_t)


def kernel(x, coefficients_vect):
    c = coefficients_vect
    lo_t = jnp.roll(c, 2048)
    d_t = jnp.roll(c, 2047) - lo_t
    b_t = lo_t - jnp.arange(_SIZE, dtype=jnp.float32) * d_t
    return _bspline_call(x, d_t, b_t)


# speedup vs baseline: 1.0253x; 1.0253x over previous
import functools

import jax
import jax.numpy as jnp
from jax import lax
from jax.experimental import pallas as pl
from jax.experimental.pallas import tpu as pltpu
from jax.experimental.pallas import tpu_sc as plsc

_SIZE = 4096
_NC = 2
_NS = 16
_NW = _NC * _NS
_L = 16
_R = 4
_NBUF = 4


def _bspline_call(x, a_t, b_t):
    b_dim, one, h, w = x.shape
    rows = b_dim * h
    rows_w = rows // _NW
    nch = rows_w // _R
    mesh = plsc.VectorSubcoreMesh(core_axis_name="c", subcore_axis_name="s")

    @functools.partial(
        pl.kernel,
        out_type=jax.ShapeDtypeStruct(x.shape, jnp.float32),
        mesh=mesh,
        scratch_types=[
            pltpu.VMEM((_SIZE,), jnp.float32),
            pltpu.VMEM((_SIZE,), jnp.float32),
            pltpu.VMEM((_NBUF, _R, 2048), jnp.float32),
            pltpu.VMEM((_NBUF, _R, 2048), jnp.float32),
        ] + [pltpu.SemaphoreType.DMA] * (2 * _NBUF),
        compiler_params=pltpu.CompilerParams(
            needs_layout_passes=False, use_tc_tiling_on_sc=True),
    )
    def k(x_hbm, a_hbm, b_hbm, out_hbm, a_v, b_v, in_v, out_v, *sems):
        wid = lax.axis_index("s") * _NC + lax.axis_index("c")
        row0 = wid * rows_w
        batch = row0 // h
        brow0 = row0 % h
        sems_in = sems[:_NBUF]
        sems_out = sems[_NBUF:]

        pltpu.sync_copy(a_hbm, a_v)
        pltpu.sync_copy(b_hbm, b_v)

        def in_cp(g, b):
            return pltpu.make_async_copy(
                x_hbm.at[batch, 0, pl.ds(brow0 + g * _R, _R), :],
                in_v.at[b], sems_in[b])

        def out_cp(g, b):
            return pltpu.make_async_copy(
                out_v.at[b],
                out_hbm.at[batch, 0, pl.ds(brow0 + g * _R, _R), :],
                sems_out[b])

        def compute(b):
            @plsc.parallel_loop(0, w, step=_L, unroll=4)
            def _vec(j):
                for r in range(_R):
                    xv = in_v[b, r, pl.ds(j, _L)]
                    t = xv * 1000.0 + 2048.0
                    t = jnp.minimum(jnp.maximum(t, 0.0), 4095.0)
                    i = t.astype(jnp.int32)
                    a = plsc.load_gather(a_v, [i])
                    bb = plsc.load_gather(b_v, [i])
                    out_v[b, r, pl.ds(j, _L)] = a * t + bb

        for b in range(_NBUF):
            in_cp(b, b).start()

        def ring(p, carry):
            for b in range(_NBUF):
                g = p * _NBUF + b
                in_cp(g, b).wait()

                @pl.when(g >= _NBUF)
                def _wait_out():
                    out_cp(g - _NBUF, b).wait()

                compute(b)
                out_cp(g, b).start()

                @pl.when(g + _NBUF < nch)
                def _next_in():
                    in_cp(g + _NBUF, b).start()

            return carry

        lax.fori_loop(0, nch // _NBUF, ring, 0)
        for b in range(_NBUF):
            out_cp(nch - _NBUF + b, b).wait()

    return k(x, a_t, b_t)


def kernel(x, coefficients_vect):
    c = coefficients_vect
    lo_t = jnp.roll(c, 2048)
    d_t = jnp.roll(c, 2047) - lo_t
    b_t = lo_t - jnp.arange(_SIZE, dtype=jnp.float32) * d_t
    return _bspline_call(x, d_t, b_t)


# speedup vs baseline: 1.0663x; 1.0399x over previous
import functools

import jax
import jax.numpy as jnp
from jax import lax
from jax.experimental import pallas as pl
from jax.experimental.pallas import tpu as pltpu
from jax.experimental.pallas import tpu_sc as plsc

_SIZE = 4096
_NC = 2
_NS = 16
_NW = _NC * _NS
_L = 16
_R = 4
_NBUF = 4


def _bspline_call(x, a_t, b_t):
    b_dim, one, h, w = x.shape
    rows = b_dim * h
    rows_w = rows // _NW
    nch = rows_w // _R
    mesh = plsc.VectorSubcoreMesh(core_axis_name="c", subcore_axis_name="s")

    @functools.partial(
        pl.kernel,
        out_type=jax.ShapeDtypeStruct(x.shape, jnp.float32),
        mesh=mesh,
        scratch_types=[
            pltpu.VMEM((_SIZE,), jnp.float32),
            pltpu.VMEM((_SIZE,), jnp.float32),
            pltpu.VMEM((_NBUF, _R, 2048), jnp.float32),
            pltpu.VMEM((_NBUF, _R, 2048), jnp.float32),
        ] + [pltpu.SemaphoreType.DMA] * (2 * _NBUF),
        compiler_params=pltpu.CompilerParams(
            needs_layout_passes=False, use_tc_tiling_on_sc=True),
    )
    def k(x_hbm, a_hbm, b_hbm, out_hbm, a_v, b_v, in_v, out_v, *sems):
        wid = lax.axis_index("s") * _NC + lax.axis_index("c")
        row0 = wid * rows_w
        batch = row0 // h
        brow0 = row0 % h
        sems_in = sems[:_NBUF]
        sems_out = sems[_NBUF:]

        pltpu.sync_copy(a_hbm, a_v)
        pltpu.sync_copy(b_hbm, b_v)

        def in_cp(g, b):
            return pltpu.make_async_copy(
                x_hbm.at[batch, 0, pl.ds(brow0 + g * _R, _R), :],
                in_v.at[b], sems_in[b])

        def out_cp(g, b):
            return pltpu.make_async_copy(
                out_v.at[b],
                out_hbm.at[batch, 0, pl.ds(brow0 + g * _R, _R), :],
                sems_out[b])

        def compute(b):
            @plsc.parallel_loop(0, w, step=_L, unroll=8)
            def _vec(j):
                for r in range(_R):
                    xv = in_v[b, r, pl.ds(j, _L)]
                    t = xv * 1000.0 + 2048.0
                    t = jnp.minimum(jnp.maximum(t, 0.0), 4095.0)
                    i = t.astype(jnp.int32)
                    a = plsc.load_gather(a_v, [i])
                    bb = plsc.load_gather(b_v, [i])
                    out_v[b, r, pl.ds(j, _L)] = a * t + bb

        for b in range(_NBUF):
            in_cp(b, b).start()

        def ring(p, carry):
            for b in range(_NBUF):
                g = p * _NBUF + b
                in_cp(g, b).wait()

                @pl.when(g >= _NBUF)
                def _wait_out():
                    out_cp(g - _NBUF, b).wait()

                compute(b)
                out_cp(g, b).start()

                @pl.when(g + _NBUF < nch)
                def _next_in():
                    in_cp(g + _NBUF, b).start()

            return carry

        lax.fori_loop(0, nch // _NBUF, ring, 0)
        for b in range(_NBUF):
            out_cp(nch - _NBUF + b, b).wait()

    return k(x, a_t, b_t)


def kernel(x, coefficients_vect):
    c = coefficients_vect
    lo_t = jnp.roll(c, 2048)
    d_t = jnp.roll(c, 2047) - lo_t
    b_t = lo_t - jnp.arange(_SIZE, dtype=jnp.float32) * d_t
    return _bspline_call(x, d_t, b_t)
